# speedup vs baseline: 18.1478x; 18.1478x over previous
import jax
import jax.numpy as jnp
from jax.experimental import pallas as pl

_W0 = 1.0
_W1 = 1.0
_W2 = 2.0
_T = 4.0
_NEG = -1e30


def _gdkd_body(ys_ref, yt_ref, out_ref):
    b = pl.program_id(0)
    j = pl.program_id(1)

    t = yt_ref[0] * (1.0 / _T)
    s = ys_ref[0] * (1.0 / _T)

    m1v = jnp.max(t, axis=0, keepdims=True)
    t_wo1 = jnp.where(t >= m1v, _NEG, t)
    m2v = jnp.max(t_wo1, axis=0, keepdims=True)
    t_wo2 = jnp.where(t_wo1 >= m2v, _NEG, t_wo1)
    thr = jnp.max(t_wo2, axis=0, keepdims=True)
    mask1 = t >= thr

    et = jnp.exp(t - m1v)
    s1_t = jnp.sum(jnp.where(mask1, et, 0.0), axis=0, keepdims=True)
    s2_t = jnp.sum(jnp.where(mask1, 0.0, et), axis=0, keepdims=True)

    smax = jnp.max(s, axis=0, keepdims=True)
    es = jnp.exp(s - smax)
    s1_s = jnp.sum(jnp.where(mask1, es, 0.0), axis=0, keepdims=True)
    s2_s = jnp.sum(jnp.where(mask1, 0.0, es), axis=0, keepdims=True)

    w = et * (t - s)
    a1 = jnp.sum(jnp.where(mask1, w, 0.0), axis=0, keepdims=True)
    a2 = jnp.sum(jnp.where(mask1, 0.0, w), axis=0, keepdims=True)

    ls1_t = jnp.log(s1_t)
    ls2_t = jnp.log(s2_t)
    lsa_t = jnp.log(s1_t + s2_t)
    ls1_s = jnp.log(s1_s)
    ls2_s = jnp.log(s2_s)
    lsa_s = jnp.log(s1_s + s2_s)

    la_t = ls1_t - lsa_t
    lb_t = ls2_t - lsa_t
    la_s = ls1_s - lsa_s
    lb_s = ls2_s - lsa_s
    high = jnp.exp(la_t) * (la_t - la_s) + jnp.exp(lb_t) * (lb_t - lb_s)

    dmax = m1v - smax
    low_top = a1 / s1_t - (ls1_t - ls1_s + dmax)
    low_other = a2 / s2_t - (ls2_t - ls2_s + dmax)

    c = _W0 * high + _W1 * low_top + _W2 * low_other
    cv = jnp.sum(c.reshape(-1, 128), axis=0, keepdims=True)

    @pl.when((b == 0) & (j == 0))
    def _init():
        out_ref[...] = jnp.zeros_like(out_ref)

    out_ref[...] += cv


def kernel(y_s, y_t):
    bsz, num_classes, h, w = y_s.shape
    p = h * w
    n = bsz * p
    pb = 1024
    ys3 = y_s.reshape(bsz, num_classes, p)
    yt3 = y_t.reshape(bsz, num_classes, p)
    acc = pl.pallas_call(
        _gdkd_body,
        grid=(bsz, p // pb),
        in_specs=[
            pl.BlockSpec((1, num_classes, pb), lambda b, j: (b, 0, j)),
            pl.BlockSpec((1, num_classes, pb), lambda b, j: (b, 0, j)),
        ],
        out_specs=pl.BlockSpec((1, 128), lambda b, j: (0, 0)),
        out_shape=jax.ShapeDtypeStruct((1, 128), jnp.float32),
    )(ys3, yt3)
    return jnp.sum(acc) * (_T * _T / n)


# speedup vs baseline: 18.9498x; 1.0442x over previous
import jax
import jax.numpy as jnp
from jax.experimental import pallas as pl

_W0 = 1.0
_W1 = 1.0
_W2 = 2.0
_T = 4.0
_NEG = -1e30


def _gdkd_body(ys_ref, yt_ref, out_ref):
    b = pl.program_id(0)
    j = pl.program_id(1)

    t = yt_ref[0] * (1.0 / _T)
    s = ys_ref[0] * (1.0 / _T)

    m1v = jnp.max(t, axis=0, keepdims=True)
    t_wo1 = jnp.where(t >= m1v, _NEG, t)
    m2v = jnp.max(t_wo1, axis=0, keepdims=True)
    t_wo2 = jnp.where(t_wo1 >= m2v, _NEG, t_wo1)
    thr = jnp.max(t_wo2, axis=0, keepdims=True)
    mask1 = t >= thr

    et = jnp.exp(t - m1v)
    s1_t = 1.0 + jnp.exp(m2v - m1v) + jnp.exp(thr - m1v)
    sa_t = jnp.sum(et, axis=0, keepdims=True)
    s2_t = sa_t - s1_t

    smax = jnp.max(s, axis=0, keepdims=True)
    es = jnp.exp(s - smax)
    s1_s = jnp.sum(jnp.where(mask1, es, 0.0), axis=0, keepdims=True)
    sa_s = jnp.sum(es, axis=0, keepdims=True)
    s2_s = sa_s - s1_s

    w = et * (t - s)
    a1 = jnp.sum(jnp.where(mask1, w, 0.0), axis=0, keepdims=True)
    aa = jnp.sum(w, axis=0, keepdims=True)
    a2 = aa - a1

    ls1_t = jnp.log(s1_t)
    ls2_t = jnp.log(s2_t)
    lsa_t = jnp.log(sa_t)
    ls1_s = jnp.log(s1_s)
    ls2_s = jnp.log(s2_s)
    lsa_s = jnp.log(sa_s)

    la_t = ls1_t - lsa_t
    lb_t = ls2_t - lsa_t
    la_s = ls1_s - lsa_s
    lb_s = ls2_s - lsa_s
    high = jnp.exp(la_t) * (la_t - la_s) + jnp.exp(lb_t) * (lb_t - lb_s)

    dmax = m1v - smax
    low_top = a1 / s1_t - (ls1_t - ls1_s + dmax)
    low_other = a2 / s2_t - (ls2_t - ls2_s + dmax)

    c = _W0 * high + _W1 * low_top + _W2 * low_other
    cv = jnp.sum(c.reshape(-1, 128), axis=0, keepdims=True)

    @pl.when((b == 0) & (j == 0))
    def _init():
        out_ref[...] = jnp.zeros_like(out_ref)

    out_ref[...] += cv


def kernel(y_s, y_t):
    bsz, num_classes, h, w = y_s.shape
    p = h * w
    n = bsz * p
    pb = 1024
    ys3 = y_s.reshape(bsz, num_classes, p)
    yt3 = y_t.reshape(bsz, num_classes, p)
    acc = pl.pallas_call(
        _gdkd_body,
        grid=(bsz, p // pb),
        in_specs=[
            pl.BlockSpec((1, num_classes, pb), lambda b, j: (b, 0, j)),
            pl.BlockSpec((1, num_classes, pb), lambda b, j: (b, 0, j)),
        ],
        out_specs=pl.BlockSpec((1, 128), lambda b, j: (0, 0)),
        out_shape=jax.ShapeDtypeStruct((1, 128), jnp.float32),
    )(ys3, yt3)
    return jnp.sum(acc) * (_T * _T / n)


# speedup vs baseline: 20.7489x; 1.0949x over previous
import jax
import jax.numpy as jnp
from jax.experimental import pallas as pl

_W0 = 1.0
_W1 = 1.0
_W2 = 2.0
_T = 4.0
_NEG = -1e30


def _gdkd_body(ys_ref, yt_ref, out_ref):
    b = pl.program_id(0)
    j = pl.program_id(1)

    t = yt_ref[0] * (1.0 / _T)
    s = ys_ref[0] * (1.0 / _T)

    m1v = jnp.max(t, axis=0, keepdims=True)
    t_wo1 = jnp.where(t >= m1v, _NEG, t)
    m2v = jnp.max(t_wo1, axis=0, keepdims=True)
    t_wo2 = jnp.where(t_wo1 >= m2v, _NEG, t_wo1)
    thr = jnp.max(t_wo2, axis=0, keepdims=True)
    mask1 = t >= thr

    et = jnp.exp(t - m1v)
    s1_t = 1.0 + jnp.exp(m2v - m1v) + jnp.exp(thr - m1v)
    sa_t = jnp.sum(et, axis=0, keepdims=True)
    s2_t = sa_t - s1_t

    smax = jnp.max(s, axis=0, keepdims=True)
    es = jnp.exp(s - smax)
    s1_s = jnp.sum(jnp.where(mask1, es, 0.0), axis=0, keepdims=True)
    sa_s = jnp.sum(es, axis=0, keepdims=True)
    s2_s = sa_s - s1_s

    w = et * (t - s)
    a1 = jnp.sum(jnp.where(mask1, w, 0.0), axis=0, keepdims=True)
    aa = jnp.sum(w, axis=0, keepdims=True)
    a2 = aa - a1

    ls1_t = jnp.log(s1_t)
    ls2_t = jnp.log(s2_t)
    lsa_t = jnp.log(sa_t)
    ls1_s = jnp.log(s1_s)
    ls2_s = jnp.log(s2_s)
    lsa_s = jnp.log(sa_s)

    la_t = ls1_t - lsa_t
    lb_t = ls2_t - lsa_t
    la_s = ls1_s - lsa_s
    lb_s = ls2_s - lsa_s
    high = jnp.exp(la_t) * (la_t - la_s) + jnp.exp(lb_t) * (lb_t - lb_s)

    dmax = m1v - smax
    low_top = a1 / s1_t - (ls1_t - ls1_s + dmax)
    low_other = a2 / s2_t - (ls2_t - ls2_s + dmax)

    c = _W0 * high + _W1 * low_top + _W2 * low_other
    cv = jnp.sum(c.reshape(-1, 128), axis=0, keepdims=True)

    @pl.when((b == 0) & (j == 0))
    def _init():
        out_ref[...] = jnp.zeros_like(out_ref)

    out_ref[...] += cv


def kernel(y_s, y_t):
    bsz, num_classes, h, w = y_s.shape
    p = h * w
    n = bsz * p
    pb = 2048
    ys3 = y_s.reshape(bsz, num_classes, p)
    yt3 = y_t.reshape(bsz, num_classes, p)
    acc = pl.pallas_call(
        _gdkd_body,
        grid=(bsz, p // pb),
        in_specs=[
            pl.BlockSpec((1, num_classes, pb), lambda b, j: (b, 0, j)),
            pl.BlockSpec((1, num_classes, pb), lambda b, j: (b, 0, j)),
        ],
        out_specs=pl.BlockSpec((1, 128), lambda b, j: (0, 0)),
        out_shape=jax.ShapeDtypeStruct((1, 128), jnp.float32),
    )(ys3, yt3)
    return jnp.sum(acc) * (_T * _T / n)


# speedup vs baseline: 54.1847x; 2.6114x over previous
import jax
import jax.numpy as jnp
from jax.experimental import pallas as pl

_W0 = 1.0
_W1 = 1.0
_W2 = 2.0
_T = 4.0
_NEG = -1e30


def _gdkd_body(ys_ref, yt_ref, out_ref):
    b = pl.program_id(0)
    j = pl.program_id(1)

    t = yt_ref[0] * (1.0 / _T)
    s = ys_ref[0] * (1.0 / _T)

    m1v = jnp.max(t, axis=0, keepdims=True)
    t_wo1 = jnp.where(t >= m1v, _NEG, t)
    m2v = jnp.max(t_wo1, axis=0, keepdims=True)
    t_wo2 = jnp.where(t_wo1 >= m2v, _NEG, t_wo1)
    thr = jnp.max(t_wo2, axis=0, keepdims=True)
    mask1 = t >= thr

    et = jnp.exp(t - m1v)
    s1_t = 1.0 + jnp.exp(m2v - m1v) + jnp.exp(thr - m1v)
    sa_t = jnp.sum(et, axis=0, keepdims=True)
    s2_t = sa_t - s1_t

    smax = jnp.max(s, axis=0, keepdims=True)
    es = jnp.exp(s - smax)
    s1_s = jnp.sum(jnp.where(mask1, es, 0.0), axis=0, keepdims=True)
    sa_s = jnp.sum(es, axis=0, keepdims=True)
    s2_s = sa_s - s1_s

    w = et * (t - s)
    a1 = jnp.sum(jnp.where(mask1, w, 0.0), axis=0, keepdims=True)
    aa = jnp.sum(w, axis=0, keepdims=True)
    a2 = aa - a1

    ls1_t = jnp.log(s1_t)
    ls2_t = jnp.log(s2_t)
    lsa_t = jnp.log(sa_t)
    ls1_s = jnp.log(s1_s)
    ls2_s = jnp.log(s2_s)
    lsa_s = jnp.log(sa_s)

    la_t = ls1_t - lsa_t
    lb_t = ls2_t - lsa_t
    la_s = ls1_s - lsa_s
    lb_s = ls2_s - lsa_s
    high = jnp.exp(la_t) * (la_t - la_s) + jnp.exp(lb_t) * (lb_t - lb_s)

    dmax = m1v - smax
    low_top = a1 / s1_t - (ls1_t - ls1_s + dmax)
    low_other = a2 / s2_t - (ls2_t - ls2_s + dmax)

    c = _W0 * high + _W1 * low_top + _W2 * low_other
    cv = jnp.sum(c[0].reshape(-1, 8, 128), axis=0)

    @pl.when((b == 0) & (j == 0))
    def _init():
        out_ref[...] = jnp.zeros_like(out_ref)

    out_ref[...] += cv


def kernel(y_s, y_t):
    bsz, num_classes, h, w = y_s.shape
    n = bsz * h * w
    bh = 16
    acc = pl.pallas_call(
        _gdkd_body,
        grid=(bsz, h // bh),
        in_specs=[
            pl.BlockSpec((1, num_classes, bh, w), lambda b, j: (b, 0, j, 0)),
            pl.BlockSpec((1, num_classes, bh, w), lambda b, j: (b, 0, j, 0)),
        ],
        out_specs=pl.BlockSpec((8, 128), lambda b, j: (0, 0)),
        out_shape=jax.ShapeDtypeStruct((8, 128), jnp.float32),
    )(y_s, y_t)
    return jnp.sum(acc) * (_T * _T / n)
